# baseline (device time: 197546 ns/iter reference)
import jax
import jax.numpy as jnp
from jax import lax
from jax.experimental import pallas as pl
from jax.experimental.pallas import tpu as pltpu

M = 8192
HALF = M // 2
HALF2 = HALF // 2
D = 4096
BLK = 256
N_CH = HALF2 // BLK
N_ZD = 6
N_SW = 2
N_XD = 5
N_FW = 3


def kernel(partial, gamma):
    x2d = partial.reshape(M, D)
    g2d = gamma.reshape(1, D)

    def body(x_ref, g_ref, out_ref, contrib_ref, ld_ref, sb_ref, a_ref,
             b_ref, ob_ref, zsend, zrecv, swsend, swrecv, xsend, xrecv,
             fwsend, fwrecv, ostore, cp_sems, asem, bsem):
        my_x = lax.axis_index("x")
        my_y = lax.axis_index("y")
        my_z = lax.axis_index("z")
        znbr = (my_x, my_y, 1 - my_z)
        xnbr = (1 - my_x, my_y, my_z)
        ynbr = (my_x, 1 - my_y, my_z)

        theirs = (1 - my_z) * HALF
        mine = my_z * HALF
        part = my_x * HALF2
        partner = (1 - my_x) * HALF2

        zchunks = [my_y * 6, my_y * 6 + 1, 2 + my_y, 4, 5, 3 - my_y]
        swin = [(1 - my_y) * 6, (1 - my_y) * 6 + 1]
        ARR = [("z", 0), ("z", 1), ("sw", 0), ("z", 2), ("sw", 1),
               ("z", 3), ("z", 4), ("z", 5)]
        chunk_of = [zchunks[0], zchunks[1], swin[0], zchunks[2], swin[1],
                    zchunks[3], zchunks[4], zchunks[5]]
        xk_of = [0, 1, None, 2, None, 3, 4, None]
        zprep_of = {0: 2, 1: 3, 3: 4, 5: 5}
        xrh_of = {2: 0, 4: 1, 6: 2, 7: 3}

        def stage_cast(k):
            cp = pltpu.make_async_copy(
                x_ref.at[pl.ds(theirs + part + zchunks[k] * BLK, BLK), :],
                ld_ref, cp_sems.at[0])
            cp.start()
            cp.wait()
            sb_ref[k % 2] = ld_ref[...].astype(jnp.bfloat16)

        def start_zsend(k):
            op = pltpu.make_async_remote_copy(
                src_ref=sb_ref.at[k % 2],
                dst_ref=contrib_ref.at[pl.ds(zchunks[k] * BLK, BLK), :],
                send_sem=zsend.at[k],
                recv_sem=zrecv.at[k],
                device_id=znbr,
                device_id_type=pl.DeviceIdType.MESH,
            )
            op.start()
            return op

        def swap_send(j):
            sl = pl.ds(zchunks[j] * BLK, BLK)
            op = pltpu.make_async_remote_copy(
                src_ref=contrib_ref.at[sl, :],
                dst_ref=contrib_ref.at[sl, :],
                send_sem=swsend.at[j],
                recv_sem=swrecv.at[j],
                device_id=ynbr,
                device_id_type=pl.DeviceIdType.MESH,
            )
            op.start()
            return op

        def forward(j):
            sl = pl.ds(partner + zchunks[j] * BLK, BLK)
            op = pltpu.make_async_remote_copy(
                src_ref=out_ref.at[sl, :],
                dst_ref=out_ref.at[sl, :],
                send_sem=fwsend.at[j],
                recv_sem=fwrecv.at[j],
                device_id=ynbr,
                device_id_type=pl.DeviceIdType.MESH,
            )
            op.start()
            return op

        def issue_loads(i):
            s = i % 2
            ac = chunk_of[i]
            la = pltpu.make_async_copy(
                x_ref.at[pl.ds(mine + part + ac * BLK, BLK), :],
                a_ref.at[s], asem.at[s])
            lb = pltpu.make_async_copy(
                contrib_ref.at[pl.ds(ac * BLK, BLK), :], b_ref.at[s],
                bsem.at[s])
            la.start()
            lb.start()
            return (la, lb)

        users = {}

        def compute_stage(c, loads):
            ac = chunk_of[c]
            k_x = xk_of[c]
            loads[0].wait()
            loads[1].wait()
            s = c % 2
            y = a_ref[s] + b_ref[s].astype(jnp.float32)
            rms = jnp.sqrt(jnp.mean(y * y, axis=-1, keepdims=True) + 1e-6)
            if c >= 2:
                for kind, op in users[c - 2]:
                    op.wait() if kind == "l" else op.wait_send()
            ob_ref[s] = (y / rms * g_ref[...]).astype(jnp.bfloat16)
            o_op = pltpu.make_async_copy(
                ob_ref.at[s], out_ref.at[pl.ds(part + ac * BLK, BLK), :],
                ostore.at[s])
            o_op.start()
            users[c] = [("l", o_op)]
            if k_x is not None:
                x_op = pltpu.make_async_remote_copy(
                    src_ref=ob_ref.at[s],
                    dst_ref=out_ref.at[pl.ds(part + ac * BLK, BLK), :],
                    send_sem=xsend.at[k_x],
                    recv_sem=xrecv.at[k_x],
                    device_id=xnbr,
                    device_id_type=pl.DeviceIdType.MESH,
                )
                x_op.start()
                users[c].append(("s", x_op))
                return x_op
            return None

        stage_cast(0)
        stage_cast(1)

        barrier = pltpu.get_barrier_semaphore()
        for nbr in (znbr, xnbr, ynbr):
            pl.semaphore_signal(barrier, inc=1, device_id=nbr,
                                device_id_type=pl.DeviceIdType.MESH)
        pl.semaphore_wait(barrier, 3)

        zops = {0: start_zsend(0), 1: start_zsend(1)}
        swops = {}
        xops = {}
        fwops = {}
        loads = {}
        for i in range(N_CH):
            kind, idx = ARR[i]
            if i in zprep_of:
                kz = zprep_of[i]
                zops[kz - 2].wait_send()
                stage_cast(kz)
                zops[kz] = start_zsend(kz)
            if kind == "z":
                zops[idx].wait_recv()
                if idx < N_SW:
                    swops[idx] = swap_send(idx)
            else:
                swops[idx].wait_recv()
            loads[i] = issue_loads(i)
            if i >= 1:
                xops_i = compute_stage(i - 1, loads[i - 1])
                if xops_i is not None:
                    xops[xk_of[i - 1]] = xops_i
            if i in xrh_of:
                j = xrh_of[i]
                xops[j].wait_recv()
                if j < N_FW:
                    fwops[j] = forward(j)
        xops_last = compute_stage(N_CH - 1, loads[N_CH - 1])
        xops[N_XD - 1].wait_recv()

        zops[N_ZD - 2].wait_send()
        zops[N_ZD - 1].wait_send()
        for j in range(N_SW):
            swops[j].wait_send()
        for j in range(N_FW):
            fwops[j].wait_send()
        for j in range(N_FW):
            fwops[j].wait_recv()
        for c in (N_CH - 2, N_CH - 1):
            for kind, op in users[c]:
                op.wait() if kind == "l" else op.wait_send()

    out, _ = pl.pallas_call(
        body,
        out_shape=(
            jax.ShapeDtypeStruct((HALF, D), jnp.bfloat16),
            jax.ShapeDtypeStruct((HALF2, D), jnp.bfloat16),
        ),
        in_specs=[
            pl.BlockSpec(memory_space=pl.ANY),
            pl.BlockSpec(memory_space=pltpu.VMEM),
        ],
        out_specs=(
            pl.BlockSpec(memory_space=pl.ANY),
            pl.BlockSpec(memory_space=pl.ANY),
        ),
        scratch_shapes=[
            pltpu.VMEM((BLK, D), jnp.float32),
            pltpu.VMEM((2, BLK, D), jnp.bfloat16),
            pltpu.VMEM((2, BLK, D), jnp.float32),
            pltpu.VMEM((2, BLK, D), jnp.bfloat16),
            pltpu.VMEM((2, BLK, D), jnp.bfloat16),
            pltpu.SemaphoreType.DMA((N_ZD,)),
            pltpu.SemaphoreType.DMA((N_ZD,)),
            pltpu.SemaphoreType.DMA((N_SW,)),
            pltpu.SemaphoreType.DMA((N_SW,)),
            pltpu.SemaphoreType.DMA((N_XD,)),
            pltpu.SemaphoreType.DMA((N_XD,)),
            pltpu.SemaphoreType.DMA((N_FW,)),
            pltpu.SemaphoreType.DMA((N_FW,)),
            pltpu.SemaphoreType.DMA((2,)),
            pltpu.SemaphoreType.DMA((1,)),
            pltpu.SemaphoreType.DMA((2,)),
            pltpu.SemaphoreType.DMA((2,)),
        ],
        compiler_params=pltpu.CompilerParams(collective_id=0),
    )(x2d, g2d)
    return out


# device time: 195056 ns/iter; 1.0128x vs baseline; 1.0128x over previous
import jax
import jax.numpy as jnp
from jax import lax
from jax.experimental import pallas as pl
from jax.experimental.pallas import tpu as pltpu

M = 8192
HALF = M // 2
HALF2 = HALF // 2
D = 4096
BLK = 256
N_CH = HALF2 // BLK
N_XD = 5
N_SW = 3


def kernel(partial, gamma):
    x2d = partial.reshape(M, D)
    g2d = gamma.reshape(1, D)

    def body(x_ref, g_ref, out_ref, contrib_ref, ld_ref, sb_ref, a_ref,
             b_ref, ob_ref, zsend, zrecv, swsend, swrecv, xsend, xrecv,
             fwsend, fwrecv, ostore, cp_sems):
        my_x = lax.axis_index("x")
        my_y = lax.axis_index("y")
        my_z = lax.axis_index("z")
        znbr = (my_x, my_y, 1 - my_z)
        xnbr = (1 - my_x, my_y, my_z)
        ynbr = (my_x, 1 - my_y, my_z)

        theirs = (1 - my_z) * HALF
        mine = my_z * HALF
        part = my_x * HALF2
        partner = (1 - my_x) * HALF2

        def a_of(k):
            return my_y * 5 + k if k < N_SW else k

        def s_of(k):
            return (1 - my_y) * 5 + k

        def stage_cast(k):
            cp = pltpu.make_async_copy(
                x_ref.at[pl.ds(theirs + part + a_of(k) * BLK, BLK), :],
                ld_ref, cp_sems.at[0])
            cp.start()
            cp.wait()
            sb_ref[k % 2] = ld_ref[...].astype(jnp.bfloat16)

        def start_zsend(k):
            op = pltpu.make_async_remote_copy(
                src_ref=sb_ref.at[k % 2],
                dst_ref=contrib_ref.at[pl.ds(a_of(k) * BLK, BLK), :],
                send_sem=zsend.at[k],
                recv_sem=zrecv.at[k],
                device_id=znbr,
                device_id_type=pl.DeviceIdType.MESH,
            )
            op.start()
            return op

        users = {}

        def compute_chunk(ac, c, k_x):
            cp_a = pltpu.make_async_copy(
                x_ref.at[pl.ds(mine + part + ac * BLK, BLK), :], a_ref,
                cp_sems.at[1])
            cp_b = pltpu.make_async_copy(
                contrib_ref.at[pl.ds(ac * BLK, BLK), :], b_ref,
                cp_sems.at[2])
            cp_a.start()
            cp_b.start()
            cp_a.wait()
            cp_b.wait()
            y = a_ref[...] + b_ref[...].astype(jnp.float32)
            rms = jnp.sqrt(jnp.mean(y * y, axis=-1, keepdims=True) + 1e-6)
            oslot = c % 2
            if c >= 2:
                for kind, op in users[c - 2]:
                    op.wait() if kind == "l" else op.wait_send()
            ob_ref[oslot] = (y / rms * g_ref[...]).astype(jnp.bfloat16)
            o_op = pltpu.make_async_copy(
                ob_ref.at[oslot], out_ref.at[pl.ds(part + ac * BLK, BLK), :],
                ostore.at[oslot])
            o_op.start()
            users[c] = [("l", o_op)]
            if k_x is not None:
                x_op = pltpu.make_async_remote_copy(
                    src_ref=ob_ref.at[oslot],
                    dst_ref=out_ref.at[pl.ds(part + ac * BLK, BLK), :],
                    send_sem=xsend.at[k_x],
                    recv_sem=xrecv.at[k_x],
                    device_id=xnbr,
                    device_id_type=pl.DeviceIdType.MESH,
                )
                x_op.start()
                users[c].append(("s", x_op))
                return x_op
            return None

        stage_cast(0)
        stage_cast(1)

        barrier = pltpu.get_barrier_semaphore()
        for nbr in (znbr, xnbr, ynbr):
            pl.semaphore_signal(barrier, inc=1, device_id=nbr,
                                device_id_type=pl.DeviceIdType.MESH)
        pl.semaphore_wait(barrier, 3)

        zops = {0: start_zsend(0), 1: start_zsend(1)}
        xops = {}
        swops = {}
        fwops = {}
        for k in range(N_XD):
            if k + 2 < N_XD:
                zops[k].wait_send()
                stage_cast(k + 2)
                zops[k + 2] = start_zsend(k + 2)
            zops[k].wait_recv()
            if k < N_SW:
                swops[k] = pltpu.make_async_remote_copy(
                    src_ref=contrib_ref.at[pl.ds(a_of(k) * BLK, BLK), :],
                    dst_ref=contrib_ref.at[pl.ds(a_of(k) * BLK, BLK), :],
                    send_sem=swsend.at[k],
                    recv_sem=swrecv.at[k],
                    device_id=ynbr,
                    device_id_type=pl.DeviceIdType.MESH,
                )
                swops[k].start()
            xops[k] = compute_chunk(a_of(k), k, k)
            if k >= 1:
                xops[k - 1].wait_recv()
                if k - 1 < N_SW:
                    fwops[k - 1] = pltpu.make_async_remote_copy(
                        src_ref=out_ref.at[
                            pl.ds(partner + a_of(k - 1) * BLK, BLK), :],
                        dst_ref=out_ref.at[
                            pl.ds(partner + a_of(k - 1) * BLK, BLK), :],
                        send_sem=fwsend.at[k - 1],
                        recv_sem=fwrecv.at[k - 1],
                        device_id=ynbr,
                        device_id_type=pl.DeviceIdType.MESH,
                    )
                    fwops[k - 1].start()
        xops[N_XD - 1].wait_recv()

        for k in range(N_SW):
            swops[k].wait_recv()
            compute_chunk(s_of(k), N_XD + k, None)

        zops[N_XD - 2].wait_send()
        zops[N_XD - 1].wait_send()
        for k in range(N_SW):
            swops[k].wait_send()
            fwops[k].wait_send()
        for k in range(N_SW):
            fwops[k].wait_recv()
        for c in (N_CH - 2, N_CH - 1):
            for kind, op in users[c]:
                op.wait() if kind == "l" else op.wait_send()

    out, _ = pl.pallas_call(
        body,
        out_shape=(
            jax.ShapeDtypeStruct((HALF, D), jnp.bfloat16),
            jax.ShapeDtypeStruct((HALF2, D), jnp.bfloat16),
        ),
        in_specs=[
            pl.BlockSpec(memory_space=pl.ANY),
            pl.BlockSpec(memory_space=pltpu.VMEM),
        ],
        out_specs=(
            pl.BlockSpec(memory_space=pl.ANY),
            pl.BlockSpec(memory_space=pl.ANY),
        ),
        scratch_shapes=[
            pltpu.VMEM((BLK, D), jnp.float32),
            pltpu.VMEM((2, BLK, D), jnp.bfloat16),
            pltpu.VMEM((BLK, D), jnp.float32),
            pltpu.VMEM((BLK, D), jnp.bfloat16),
            pltpu.VMEM((2, BLK, D), jnp.bfloat16),
            pltpu.SemaphoreType.DMA((N_XD,)),
            pltpu.SemaphoreType.DMA((N_XD,)),
            pltpu.SemaphoreType.DMA((N_SW,)),
            pltpu.SemaphoreType.DMA((N_SW,)),
            pltpu.SemaphoreType.DMA((N_XD,)),
            pltpu.SemaphoreType.DMA((N_XD,)),
            pltpu.SemaphoreType.DMA((N_SW,)),
            pltpu.SemaphoreType.DMA((N_SW,)),
            pltpu.SemaphoreType.DMA((2,)),
            pltpu.SemaphoreType.DMA((3,)),
        ],
        compiler_params=pltpu.CompilerParams(collective_id=0),
    )(x2d, g2d)
    return out


# device time: 187683 ns/iter; 1.0526x vs baseline; 1.0393x over previous
import jax
import jax.numpy as jnp
from jax import lax
from jax.experimental import pallas as pl
from jax.experimental.pallas import tpu as pltpu

M = 8192
HALF = M // 2
HALF2 = HALF // 2
D = 4096
BLK = 128
N_CH = HALF2 // BLK
N_XD = 11
N_SW = 5


def kernel(partial, gamma):
    x2d = partial.reshape(M, D)
    g2d = gamma.reshape(1, D)

    def body(x_ref, g_ref, out_ref, contrib_ref, ld_ref, sb_ref, a_ref,
             b_ref, ob_ref, zsend, zrecv, swsend, swrecv, xsend, xrecv,
             fwsend, fwrecv, ostore, cp_sems):
        my_x = lax.axis_index("x")
        my_y = lax.axis_index("y")
        my_z = lax.axis_index("z")
        znbr = (my_x, my_y, 1 - my_z)
        xnbr = (1 - my_x, my_y, my_z)
        ynbr = (my_x, 1 - my_y, my_z)

        theirs = (1 - my_z) * HALF
        mine = my_z * HALF
        part = my_x * HALF2
        partner = (1 - my_x) * HALF2

        def a_of(k):
            return my_y * N_XD + k if k < N_SW else k

        def s_of(k):
            return (1 - my_y) * N_XD + k

        def stage_cast(k):
            cp = pltpu.make_async_copy(
                x_ref.at[pl.ds(theirs + part + a_of(k) * BLK, BLK), :],
                ld_ref, cp_sems.at[0])
            cp.start()
            cp.wait()
            sb_ref[k % 2] = ld_ref[...].astype(jnp.bfloat16)

        def start_zsend(k):
            op = pltpu.make_async_remote_copy(
                src_ref=sb_ref.at[k % 2],
                dst_ref=contrib_ref.at[pl.ds(a_of(k) * BLK, BLK), :],
                send_sem=zsend.at[k],
                recv_sem=zrecv.at[k],
                device_id=znbr,
                device_id_type=pl.DeviceIdType.MESH,
            )
            op.start()
            return op

        users = {}

        def compute_chunk(ac, c, k_x):
            cp_a = pltpu.make_async_copy(
                x_ref.at[pl.ds(mine + part + ac * BLK, BLK), :], a_ref,
                cp_sems.at[1])
            cp_b = pltpu.make_async_copy(
                contrib_ref.at[pl.ds(ac * BLK, BLK), :], b_ref,
                cp_sems.at[2])
            cp_a.start()
            cp_b.start()
            cp_a.wait()
            cp_b.wait()
            y = a_ref[...] + b_ref[...].astype(jnp.float32)
            rms = jnp.sqrt(jnp.mean(y * y, axis=-1, keepdims=True) + 1e-6)
            oslot = c % 2
            if c >= 2:
                for kind, op in users[c - 2]:
                    op.wait() if kind == "l" else op.wait_send()
            ob_ref[oslot] = (y / rms * g_ref[...]).astype(jnp.bfloat16)
            o_op = pltpu.make_async_copy(
                ob_ref.at[oslot], out_ref.at[pl.ds(part + ac * BLK, BLK), :],
                ostore.at[oslot])
            o_op.start()
            users[c] = [("l", o_op)]
            if k_x is not None:
                x_op = pltpu.make_async_remote_copy(
                    src_ref=ob_ref.at[oslot],
                    dst_ref=out_ref.at[pl.ds(part + ac * BLK, BLK), :],
                    send_sem=xsend.at[k_x],
                    recv_sem=xrecv.at[k_x],
                    device_id=xnbr,
                    device_id_type=pl.DeviceIdType.MESH,
                )
                x_op.start()
                users[c].append(("s", x_op))
                return x_op
            return None

        stage_cast(0)
        stage_cast(1)

        barrier = pltpu.get_barrier_semaphore()
        for nbr in (znbr, xnbr, ynbr):
            pl.semaphore_signal(barrier, inc=1, device_id=nbr,
                                device_id_type=pl.DeviceIdType.MESH)
        pl.semaphore_wait(barrier, 3)

        zops = {0: start_zsend(0), 1: start_zsend(1)}
        xops = {}
        swops = {}
        fwops = {}
        for k in range(N_XD):
            if k + 2 < N_XD:
                zops[k].wait_send()
                stage_cast(k + 2)
                zops[k + 2] = start_zsend(k + 2)
            zops[k].wait_recv()
            if k < N_SW:
                swops[k] = pltpu.make_async_remote_copy(
                    src_ref=contrib_ref.at[pl.ds(a_of(k) * BLK, BLK), :],
                    dst_ref=contrib_ref.at[pl.ds(a_of(k) * BLK, BLK), :],
                    send_sem=swsend.at[k],
                    recv_sem=swrecv.at[k],
                    device_id=ynbr,
                    device_id_type=pl.DeviceIdType.MESH,
                )
                swops[k].start()
            xops[k] = compute_chunk(a_of(k), k, k)
            if k >= 1:
                xops[k - 1].wait_recv()
                if k - 1 < N_SW:
                    fwops[k - 1] = pltpu.make_async_remote_copy(
                        src_ref=out_ref.at[
                            pl.ds(partner + a_of(k - 1) * BLK, BLK), :],
                        dst_ref=out_ref.at[
                            pl.ds(partner + a_of(k - 1) * BLK, BLK), :],
                        send_sem=fwsend.at[k - 1],
                        recv_sem=fwrecv.at[k - 1],
                        device_id=ynbr,
                        device_id_type=pl.DeviceIdType.MESH,
                    )
                    fwops[k - 1].start()
        xops[N_XD - 1].wait_recv()

        for k in range(N_SW):
            swops[k].wait_recv()
            compute_chunk(s_of(k), N_XD + k, None)

        zops[N_XD - 2].wait_send()
        zops[N_XD - 1].wait_send()
        for k in range(N_SW):
            swops[k].wait_send()
            fwops[k].wait_send()
        for k in range(N_SW):
            fwops[k].wait_recv()
        for c in (N_CH - 2, N_CH - 1):
            for kind, op in users[c]:
                op.wait() if kind == "l" else op.wait_send()

    out, _ = pl.pallas_call(
        body,
        out_shape=(
            jax.ShapeDtypeStruct((HALF, D), jnp.bfloat16),
            jax.ShapeDtypeStruct((HALF2, D), jnp.bfloat16),
        ),
        in_specs=[
            pl.BlockSpec(memory_space=pl.ANY),
            pl.BlockSpec(memory_space=pltpu.VMEM),
        ],
        out_specs=(
            pl.BlockSpec(memory_space=pl.ANY),
            pl.BlockSpec(memory_space=pl.ANY),
        ),
        scratch_shapes=[
            pltpu.VMEM((BLK, D), jnp.float32),
            pltpu.VMEM((2, BLK, D), jnp.bfloat16),
            pltpu.VMEM((BLK, D), jnp.float32),
            pltpu.VMEM((BLK, D), jnp.bfloat16),
            pltpu.VMEM((2, BLK, D), jnp.bfloat16),
            pltpu.SemaphoreType.DMA((N_XD,)),
            pltpu.SemaphoreType.DMA((N_XD,)),
            pltpu.SemaphoreType.DMA((N_SW,)),
            pltpu.SemaphoreType.DMA((N_SW,)),
            pltpu.SemaphoreType.DMA((N_XD,)),
            pltpu.SemaphoreType.DMA((N_XD,)),
            pltpu.SemaphoreType.DMA((N_SW,)),
            pltpu.SemaphoreType.DMA((N_SW,)),
            pltpu.SemaphoreType.DMA((2,)),
            pltpu.SemaphoreType.DMA((3,)),
        ],
        compiler_params=pltpu.CompilerParams(collective_id=0),
    )(x2d, g2d)
    return out


# device time: 184504 ns/iter; 1.0707x vs baseline; 1.0172x over previous
import jax
import jax.numpy as jnp
from jax import lax
from jax.experimental import pallas as pl
from jax.experimental.pallas import tpu as pltpu

M = 8192
HALF = M // 2
HALF2 = HALF // 2
D = 4096
BLK = 128
N_CH = HALF2 // BLK
N_XD = 11
N_SW = 5


def kernel(partial, gamma):
    x2d = partial.reshape(M, D)
    g2d = gamma.reshape(1, D)

    def body(x_ref, g_ref, out_ref, contrib_ref, ld_ref, sb_ref, a_ref,
             b_ref, ob_ref, zsend, zrecv, swsend, swrecv, xsend, xrecv,
             fwsend, fwrecv, ostore, cp_sems):
        my_x = lax.axis_index("x")
        my_y = lax.axis_index("y")
        my_z = lax.axis_index("z")
        znbr = (my_x, my_y, 1 - my_z)
        xnbr = (1 - my_x, my_y, my_z)
        ynbr = (my_x, 1 - my_y, my_z)

        theirs = (1 - my_z) * HALF
        mine = my_z * HALF
        part = my_x * HALF2
        partner = (1 - my_x) * HALF2

        def a_of(k):
            return my_y * N_XD + k if k < N_SW else k

        def s_of(k):
            return (1 - my_y) * N_XD + k

        def stage_cast(k):
            cp = pltpu.make_async_copy(
                x_ref.at[pl.ds(theirs + part + a_of(k) * BLK, BLK), :],
                ld_ref, cp_sems.at[0])
            cp.start()
            cp.wait()
            sb_ref[k % 2] = ld_ref[...].astype(jnp.bfloat16)

        def start_zsend(k):
            op = pltpu.make_async_remote_copy(
                src_ref=sb_ref.at[k % 2],
                dst_ref=contrib_ref.at[pl.ds(a_of(k) * BLK, BLK), :],
                send_sem=zsend.at[k],
                recv_sem=zrecv.at[k],
                device_id=znbr,
                device_id_type=pl.DeviceIdType.MESH,
            )
            op.start()
            return op

        users = {}

        def compute_chunk(ac, c, k_x):
            cp_a = pltpu.make_async_copy(
                x_ref.at[pl.ds(mine + part + ac * BLK, BLK), :], a_ref,
                cp_sems.at[1])
            cp_a.start()
            cp_a.wait()
            y = (a_ref[...]
                 + contrib_ref[pl.ds(ac * BLK, BLK), :].astype(jnp.float32))
            rms = jnp.sqrt(jnp.mean(y * y, axis=-1, keepdims=True) + 1e-6)
            oslot = c % 2
            if c >= 2:
                for kind, op in users[c - 2]:
                    op.wait() if kind == "l" else op.wait_send()
            ob_ref[oslot] = (y / rms * g_ref[...]).astype(jnp.bfloat16)
            o_op = pltpu.make_async_copy(
                ob_ref.at[oslot], out_ref.at[pl.ds(part + ac * BLK, BLK), :],
                ostore.at[oslot])
            o_op.start()
            users[c] = [("l", o_op)]
            if k_x is not None:
                x_op = pltpu.make_async_remote_copy(
                    src_ref=ob_ref.at[oslot],
                    dst_ref=out_ref.at[pl.ds(part + ac * BLK, BLK), :],
                    send_sem=xsend.at[k_x],
                    recv_sem=xrecv.at[k_x],
                    device_id=xnbr,
                    device_id_type=pl.DeviceIdType.MESH,
                )
                x_op.start()
                users[c].append(("s", x_op))
                return x_op
            return None

        stage_cast(0)
        stage_cast(1)

        barrier = pltpu.get_barrier_semaphore()
        for nbr in (znbr, xnbr, ynbr):
            pl.semaphore_signal(barrier, inc=1, device_id=nbr,
                                device_id_type=pl.DeviceIdType.MESH)
        pl.semaphore_wait(barrier, 3)

        zops = {0: start_zsend(0), 1: start_zsend(1)}
        xops = {}
        swops = {}
        fwops = {}
        for k in range(N_XD):
            if k + 2 < N_XD:
                zops[k].wait_send()
                stage_cast(k + 2)
                zops[k + 2] = start_zsend(k + 2)
            zops[k].wait_recv()
            if k < N_SW:
                swops[k] = pltpu.make_async_remote_copy(
                    src_ref=contrib_ref.at[pl.ds(a_of(k) * BLK, BLK), :],
                    dst_ref=contrib_ref.at[pl.ds(a_of(k) * BLK, BLK), :],
                    send_sem=swsend.at[k],
                    recv_sem=swrecv.at[k],
                    device_id=ynbr,
                    device_id_type=pl.DeviceIdType.MESH,
                )
                swops[k].start()
            xops[k] = compute_chunk(a_of(k), k, k)
            if k >= 1:
                xops[k - 1].wait_recv()
                if k - 1 < N_SW:
                    fwops[k - 1] = pltpu.make_async_remote_copy(
                        src_ref=out_ref.at[
                            pl.ds(partner + a_of(k - 1) * BLK, BLK), :],
                        dst_ref=out_ref.at[
                            pl.ds(partner + a_of(k - 1) * BLK, BLK), :],
                        send_sem=fwsend.at[k - 1],
                        recv_sem=fwrecv.at[k - 1],
                        device_id=ynbr,
                        device_id_type=pl.DeviceIdType.MESH,
                    )
                    fwops[k - 1].start()
        xops[N_XD - 1].wait_recv()

        for k in range(N_SW):
            swops[k].wait_recv()
            compute_chunk(s_of(k), N_XD + k, None)

        zops[N_XD - 2].wait_send()
        zops[N_XD - 1].wait_send()
        for k in range(N_SW):
            swops[k].wait_send()
            fwops[k].wait_send()
        for k in range(N_SW):
            fwops[k].wait_recv()
        for c in (N_CH - 2, N_CH - 1):
            for kind, op in users[c]:
                op.wait() if kind == "l" else op.wait_send()

    out = pl.pallas_call(
        body,
        out_shape=jax.ShapeDtypeStruct((HALF, D), jnp.bfloat16),
        in_specs=[
            pl.BlockSpec(memory_space=pl.ANY),
            pl.BlockSpec(memory_space=pltpu.VMEM),
        ],
        out_specs=pl.BlockSpec(memory_space=pl.ANY),
        scratch_shapes=[
            pltpu.VMEM((HALF2, D), jnp.bfloat16),
            pltpu.VMEM((BLK, D), jnp.float32),
            pltpu.VMEM((2, BLK, D), jnp.bfloat16),
            pltpu.VMEM((BLK, D), jnp.float32),
            pltpu.VMEM((BLK, D), jnp.bfloat16),
            pltpu.VMEM((2, BLK, D), jnp.bfloat16),
            pltpu.SemaphoreType.DMA((N_XD,)),
            pltpu.SemaphoreType.DMA((N_XD,)),
            pltpu.SemaphoreType.DMA((N_SW,)),
            pltpu.SemaphoreType.DMA((N_SW,)),
            pltpu.SemaphoreType.DMA((N_XD,)),
            pltpu.SemaphoreType.DMA((N_XD,)),
            pltpu.SemaphoreType.DMA((N_SW,)),
            pltpu.SemaphoreType.DMA((N_SW,)),
            pltpu.SemaphoreType.DMA((2,)),
            pltpu.SemaphoreType.DMA((3,)),
        ],
        compiler_params=pltpu.CompilerParams(collective_id=0),
    )(x2d, g2d)
    return out


# device time: 176436 ns/iter; 1.1196x vs baseline; 1.0457x over previous
import jax
import jax.numpy as jnp
from jax import lax
from jax.experimental import pallas as pl
from jax.experimental.pallas import tpu as pltpu

M = 8192
HALF = M // 2
HALF2 = HALF // 2
D = 4096
BLK = 128
N_CH = HALF2 // BLK
N_XD = 11
N_SW = 5


def kernel(partial, gamma):
    x2d = partial.reshape(M, D)
    g2d = gamma.reshape(1, D)

    def body(x_ref, g_ref, out_ref, contrib_ref, ld_ref, sb_ref, a_ref,
             ob_ref, zsend, zrecv, swsend, swrecv, xsend, xrecv,
             fwsend, fwrecv, ostore, cp_sems):
        my_x = lax.axis_index("x")
        my_y = lax.axis_index("y")
        my_z = lax.axis_index("z")
        znbr = (my_x, my_y, 1 - my_z)
        xnbr = (1 - my_x, my_y, my_z)
        ynbr = (my_x, 1 - my_y, my_z)

        theirs = (1 - my_z) * HALF
        mine = my_z * HALF
        part = my_x * HALF2
        partner = (1 - my_x) * HALF2

        def a_of(k):
            return my_y * N_XD + k if k < N_SW else k

        def s_of(k):
            return (1 - my_y) * N_XD + k

        def stage_cast(k):
            cp = pltpu.make_async_copy(
                x_ref.at[pl.ds(theirs + part + a_of(k) * BLK, BLK), :],
                ld_ref, cp_sems.at[0])
            cp.start()
            cp.wait()
            sb_ref[k % 2] = ld_ref[...].astype(jnp.bfloat16)

        def start_zsend(k):
            op = pltpu.make_async_remote_copy(
                src_ref=sb_ref.at[k % 2],
                dst_ref=contrib_ref.at[pl.ds(a_of(k) * BLK, BLK), :],
                send_sem=zsend.at[k],
                recv_sem=zrecv.at[k],
                device_id=znbr,
                device_id_type=pl.DeviceIdType.MESH,
            )
            op.start()
            return op

        users = {}
        aops = {}

        def comp_id(c):
            return a_of(c) if c < N_XD else s_of(c - N_XD)

        def start_aload(c):
            s = c % 2
            op = pltpu.make_async_copy(
                x_ref.at[pl.ds(mine + part + comp_id(c) * BLK, BLK), :],
                a_ref.at[s], cp_sems.at[1 + s])
            op.start()
            aops[c] = op

        def compute_chunk(ac, c, k_x):
            aops[c].wait()
            y = (a_ref[c % 2]
                 + contrib_ref[pl.ds(ac * BLK, BLK), :].astype(jnp.float32))
            rms = jnp.sqrt(jnp.mean(y * y, axis=-1, keepdims=True) + 1e-6)
            oslot = c % 2
            if c >= 2:
                for kind, op in users[c - 2]:
                    op.wait() if kind == "l" else op.wait_send()
            ob_ref[oslot] = (y / rms * g_ref[...]).astype(jnp.bfloat16)
            if c + 2 < N_CH:
                start_aload(c + 2)
            o_op = pltpu.make_async_copy(
                ob_ref.at[oslot], out_ref.at[pl.ds(part + ac * BLK, BLK), :],
                ostore.at[oslot])
            o_op.start()
            users[c] = [("l", o_op)]
            if k_x is not None:
                x_op = pltpu.make_async_remote_copy(
                    src_ref=ob_ref.at[oslot],
                    dst_ref=out_ref.at[pl.ds(part + ac * BLK, BLK), :],
                    send_sem=xsend.at[k_x],
                    recv_sem=xrecv.at[k_x],
                    device_id=xnbr,
                    device_id_type=pl.DeviceIdType.MESH,
                )
                x_op.start()
                users[c].append(("s", x_op))
                return x_op
            return None

        stage_cast(0)
        stage_cast(1)
        start_aload(0)
        start_aload(1)

        barrier = pltpu.get_barrier_semaphore()
        for nbr in (znbr, xnbr, ynbr):
            pl.semaphore_signal(barrier, inc=1, device_id=nbr,
                                device_id_type=pl.DeviceIdType.MESH)
        pl.semaphore_wait(barrier, 3)

        zops = {0: start_zsend(0), 1: start_zsend(1)}
        xops = {}
        swops = {}
        fwops = {}
        for k in range(N_XD):
            if k + 2 < N_XD:
                zops[k].wait_send()
                stage_cast(k + 2)
                zops[k + 2] = start_zsend(k + 2)
            zops[k].wait_recv()
            if k < N_SW:
                swops[k] = pltpu.make_async_remote_copy(
                    src_ref=contrib_ref.at[pl.ds(a_of(k) * BLK, BLK), :],
                    dst_ref=contrib_ref.at[pl.ds(a_of(k) * BLK, BLK), :],
                    send_sem=swsend.at[k],
                    recv_sem=swrecv.at[k],
                    device_id=ynbr,
                    device_id_type=pl.DeviceIdType.MESH,
                )
                swops[k].start()
            xops[k] = compute_chunk(a_of(k), k, k)
            if k >= 1:
                xops[k - 1].wait_recv()
                if k - 1 < N_SW:
                    fwops[k - 1] = pltpu.make_async_remote_copy(
                        src_ref=out_ref.at[
                            pl.ds(partner + a_of(k - 1) * BLK, BLK), :],
                        dst_ref=out_ref.at[
                            pl.ds(partner + a_of(k - 1) * BLK, BLK), :],
                        send_sem=fwsend.at[k - 1],
                        recv_sem=fwrecv.at[k - 1],
                        device_id=ynbr,
                        device_id_type=pl.DeviceIdType.MESH,
                    )
                    fwops[k - 1].start()
        xops[N_XD - 1].wait_recv()

        for k in range(N_SW):
            swops[k].wait_recv()
            compute_chunk(s_of(k), N_XD + k, None)

        zops[N_XD - 2].wait_send()
        zops[N_XD - 1].wait_send()
        for k in range(N_SW):
            swops[k].wait_send()
            fwops[k].wait_send()
        for k in range(N_SW):
            fwops[k].wait_recv()
        for c in (N_CH - 2, N_CH - 1):
            for kind, op in users[c]:
                op.wait() if kind == "l" else op.wait_send()

    out = pl.pallas_call(
        body,
        out_shape=jax.ShapeDtypeStruct((HALF, D), jnp.bfloat16),
        in_specs=[
            pl.BlockSpec(memory_space=pl.ANY),
            pl.BlockSpec(memory_space=pltpu.VMEM),
        ],
        out_specs=pl.BlockSpec(memory_space=pl.ANY),
        scratch_shapes=[
            pltpu.VMEM((HALF2, D), jnp.bfloat16),
            pltpu.VMEM((BLK, D), jnp.float32),
            pltpu.VMEM((2, BLK, D), jnp.bfloat16),
            pltpu.VMEM((2, BLK, D), jnp.float32),
            pltpu.VMEM((2, BLK, D), jnp.bfloat16),
            pltpu.SemaphoreType.DMA((N_XD,)),
            pltpu.SemaphoreType.DMA((N_XD,)),
            pltpu.SemaphoreType.DMA((N_SW,)),
            pltpu.SemaphoreType.DMA((N_SW,)),
            pltpu.SemaphoreType.DMA((N_XD,)),
            pltpu.SemaphoreType.DMA((N_XD,)),
            pltpu.SemaphoreType.DMA((N_SW,)),
            pltpu.SemaphoreType.DMA((N_SW,)),
            pltpu.SemaphoreType.DMA((2,)),
            pltpu.SemaphoreType.DMA((3,)),
        ],
        compiler_params=pltpu.CompilerParams(collective_id=0),
    )(x2d, g2d)
    return out
